# Initial kernel scaffold; baseline (speedup 1.0000x reference)
import jax
import jax.numpy as jnp
from jax.experimental import pallas as pl


def kernel(ids, feats, adj, W_x1, b_x1, W_n1, b_n1, W_x2, b_x2, W_n2, b_n2, fc_W, fc_b):
    raise NotImplementedError("write your pallas kernel here")


# speedup vs baseline: 6.3045x; 6.3045x over previous
import functools

import jax
import jax.numpy as jnp
from jax import lax
from jax.experimental import pallas as pl
from jax.experimental.pallas import tpu as pltpu
from jax.experimental.pallas import tpu_sc as plsc

N_NODES = 10000
D = 128
MAX_DEG = 32
BATCH = 1024
F1 = 25
F2 = 10
N_CLASSES = 32

NC = 2
NS = 16
NW = NC * NS
L = 16


def _tc_precompute(feats, W_x1, b_x1, W_n1, b_n1):
    def body(f, wx, bx, wn, bn, gx_out, q_out):
        x = f[...]
        gx_out[...] = jnp.maximum(
            jnp.dot(x, wx[...], preferred_element_type=jnp.float32) + bx[...], 0.0)
        q_out[...] = jnp.dot(x, wn[...], preferred_element_type=jnp.float32) + bn[...]

    return pl.pallas_call(
        body,
        out_shape=(jax.ShapeDtypeStruct((N_NODES, D), jnp.float32),
                   jax.ShapeDtypeStruct((N_NODES, D), jnp.float32)),
    )(feats, W_x1, b_x1.reshape(1, D), W_n1, b_n1.reshape(1, D))


_K2_NPT = 320
_K2_C = 8
_K2_NCH = 40


def _sc_layer1_table(adj10, q):
    mesh = plsc.VectorSubcoreMesh(core_axis_name="c", subcore_axis_name="s")

    @functools.partial(
        pl.kernel,
        out_type=jax.ShapeDtypeStruct((N_NODES, D), jnp.float32),
        mesh=mesh,
        scratch_types=[
            pltpu.VMEM((_K2_C * F2,), jnp.int32),
            pltpu.VMEM((_K2_C * F2, D), jnp.float32),
            pltpu.VMEM((_K2_C, D), jnp.float32),
            pltpu.SemaphoreType.DMA,
        ],
    )
    def k2(adj10_hbm, q_hbm, gn_hbm, idxbuf, rowsbuf, outbuf, sem):
        wid = lax.axis_index("s") * NC + lax.axis_index("c")

        def chunk(ch, carry):
            base = wid * _K2_NPT + ch * _K2_C
            cs = jnp.minimum(base, N_NODES - _K2_C)
            pltpu.sync_copy(adj10_hbm.at[pl.ds(cs * F2, _K2_C * F2)], idxbuf)
            pltpu.async_copy(q_hbm.at[idxbuf], rowsbuf, sem).wait()
            for i in range(_K2_C):
                for lb in range(D // L):
                    sl = slice(lb * L, (lb + 1) * L)
                    acc = rowsbuf[i * F2, sl]
                    for j in range(1, F2):
                        acc = acc + rowsbuf[i * F2 + j, sl]
                    outbuf[i, sl] = jnp.maximum(acc * (1.0 / F2), 0.0)
            pltpu.sync_copy(outbuf, gn_hbm.at[pl.ds(cs, _K2_C)])
            return carry

        lax.fori_loop(0, _K2_NCH, chunk, 0)

    return k2(adj10, q)


_K3_SPW = BATCH // NW
_K3_C = 4
_K3_NCH = _K3_SPW // _K3_C


def _sc_seed_aggregate(ids, adj, q, gx, gn):
    mesh = plsc.VectorSubcoreMesh(core_axis_name="c", subcore_axis_name="s")
    S = jax.ShapeDtypeStruct((BATCH, D), jnp.float32)

    @functools.partial(
        pl.kernel,
        out_type=(S, S, S, S),
        mesh=mesh,
        scratch_types=[
            pltpu.VMEM((_K3_SPW,), jnp.int32),
            pltpu.VMEM((_K3_SPW, 128), jnp.int32),
            pltpu.VMEM((_K3_C * MAX_DEG,), jnp.int32),
            pltpu.VMEM((_K3_C * MAX_DEG, D), jnp.float32),
            pltpu.VMEM((_K3_SPW, D), jnp.float32),
            pltpu.VMEM((_K3_SPW, D), jnp.float32),
            pltpu.VMEM((_K3_SPW, D), jnp.float32),
            pltpu.VMEM((_K3_SPW, D), jnp.float32),
            pltpu.SemaphoreType.DMA,
        ],
    )
    def k3(ids_hbm, adj_hbm, q_hbm, gx_hbm, gn_hbm,
           g0top_hbm, g0bot_hbm, mgx_hbm, mgn_hbm,
           sbuf, adjbuf, idxbuf, rowsbuf, botbuf, mgxbuf, mgnbuf, topbuf, sem):
        wid = lax.axis_index("s") * NC + lax.axis_index("c")
        base = wid * _K3_SPW

        pltpu.sync_copy(ids_hbm.at[pl.ds(base, _K3_SPW)], sbuf)
        pltpu.async_copy(adj_hbm.at[sbuf], adjbuf, sem).wait()
        pltpu.async_copy(gx_hbm.at[sbuf], topbuf, sem).wait()
        pltpu.sync_copy(topbuf, g0top_hbm.at[pl.ds(base, _K3_SPW)])

        def reduce_into(dstbuf, s0, relu):
            for i in range(_K3_C):
                for lb in range(D // L):
                    sl = slice(lb * L, (lb + 1) * L)
                    acc = rowsbuf[i * MAX_DEG, sl]
                    for j in range(1, F1):
                        acc = acc + rowsbuf[i * MAX_DEG + j, sl]
                    acc = acc * (1.0 / F1)
                    if relu:
                        acc = jnp.maximum(acc, 0.0)
                    dstbuf[s0 + i, sl] = acc

        def chunk(ch, carry):
            s0 = ch * _K3_C
            for i in range(_K3_C):
                idxbuf[pl.ds(i * MAX_DEG, L)] = adjbuf[s0 + i, 0:L]
                idxbuf[pl.ds(i * MAX_DEG + L, L)] = adjbuf[s0 + i, L:2 * L]
            pltpu.async_copy(q_hbm.at[idxbuf], rowsbuf, sem).wait()
            reduce_into(botbuf, s0, relu=True)
            pltpu.async_copy(gx_hbm.at[idxbuf], rowsbuf, sem).wait()
            reduce_into(mgxbuf, s0, relu=False)
            pltpu.async_copy(gn_hbm.at[idxbuf], rowsbuf, sem).wait()
            reduce_into(mgnbuf, s0, relu=False)
            return carry

        lax.fori_loop(0, _K3_NCH, chunk, 0)
        pltpu.sync_copy(botbuf, g0bot_hbm.at[pl.ds(base, _K3_SPW)])
        pltpu.sync_copy(mgxbuf, mgx_hbm.at[pl.ds(base, _K3_SPW)])
        pltpu.sync_copy(mgnbuf, mgn_hbm.at[pl.ds(base, _K3_SPW)])

    return k3(ids, adj, q, gx, gn)


def _tc_head(g0top, g0bot, mgx, mgn, W_x2, b_x2, W_n2, b_n2, fc_W, fc_b):
    def body(t, b, mx, mn, wx, bx, wn, bn, fw, fb, out):
        dot = functools.partial(jnp.dot, preferred_element_type=jnp.float32)
        A = jnp.maximum(dot(t[...], wx[0:D, :]) + dot(b[...], wx[D:2 * D, :])
                        + bx[...], 0.0)
        B = jnp.maximum(dot(mx[...], wn[0:D, :]) + dot(mn[...], wn[D:2 * D, :])
                        + bn[...], 0.0)
        nrm = jnp.sqrt(jnp.sum(A * A, axis=1, keepdims=True)
                       + jnp.sum(B * B, axis=1, keepdims=True))
        nrm = jnp.maximum(nrm, 1e-12)
        out[...] = (dot(A, fw[0:D, :]) + dot(B, fw[D:2 * D, :])) / nrm + fb[...]

    return pl.pallas_call(
        body,
        out_shape=jax.ShapeDtypeStruct((BATCH, N_CLASSES), jnp.float32),
    )(g0top, g0bot, mgx, mgn, W_x2, b_x2.reshape(1, D), W_n2,
      b_n2.reshape(1, D), fc_W, fc_b.reshape(1, N_CLASSES))


def kernel(ids, feats, adj, W_x1, b_x1, W_n1, b_n1, W_x2, b_x2, W_n2, b_n2,
           fc_W, fc_b):
    gx, q = _tc_precompute(feats, W_x1, b_x1, W_n1, b_n1)
    adj10 = adj[:, :F2].reshape(-1)
    gn = _sc_layer1_table(adj10, q)
    adj_pad = jnp.pad(adj, ((0, 0), (0, 128 - MAX_DEG)))
    g0top, g0bot, mgx, mgn = _sc_seed_aggregate(ids, adj_pad, q, gx, gn)
    return _tc_head(g0top, g0bot, mgx, mgn, W_x2, b_x2, W_n2, b_n2, fc_W, fc_b)
